# baseline (speedup 1.0000x reference)
import functools

import numpy as np

import jax
import jax.numpy as jnp
from jax import lax
from jax.experimental import pallas as pl
from jax.experimental.pallas import tpu as pltpu
from jax.experimental.pallas import tpu_sc as plsc

N = 10000
NPAD = 10240
D = 128
EH = 160000
NC = 2
NS = 16
EPT = EH // NS
CH = 80
NCHUNK = EPT // CH
RPT = NPAD // NS


def _bcast_lane(v16, lane):
    idx = jnp.full((16, 1), lane, jnp.int32)
    dnums = lax.GatherDimensionNumbers(
        offset_dims=(), collapsed_slice_dims=(0,), start_index_map=(0,))
    return lax.gather(v16, idx, dnums, (1,),
                      mode=lax.GatherScatterMode.PROMISE_IN_BOUNDS)


def _rsqrt16(x):
    i = lax.bitcast_convert_type(x, jnp.int32)
    i = jnp.int32(0x5F3759DF) - lax.shift_right_logical(i, 1)
    y = lax.bitcast_convert_type(i, jnp.float32)
    for _ in range(3):
        y = y * (1.5 - 0.5 * x * y * y)
    return jnp.where(x > 0.5, y, 0.0)


def _norm_body(rows_hbm, dinv_hbm,
               acc_sh, row_v, ones_v, zb_v, degb_v, dloc_v):
    c = lax.axis_index("c")
    s = lax.axis_index("s")
    zero16 = jnp.zeros((16,), jnp.float32)
    one16 = jnp.ones((16,), jnp.float32)

    def init_row(r, _):
        zb_v[pl.ds(r * 16, 16)] = zero16
        return 0
    lax.fori_loop(0, RPT // 16, init_row, 0)

    def init_ones(r, _):
        ones_v[pl.ds(r * 16, 16)] = one16
        return 0
    lax.fori_loop(0, CH // 16, init_ones, 0)

    for h in range(2):
        pltpu.sync_copy(zb_v, acc_sh.at[pl.ds(s * RPT, RPT)])
        plsc.subcore_barrier()

        off = (c * 2 + h) * EH + s * EPT
        pltpu.sync_copy(rows_hbm.at[pl.ds(off, EPT)], row_v)

        def deg_chunk(k, _):
            rc = row_v.at[pl.ds(k * CH, CH)]
            pltpu.sync_copy(ones_v, acc_sh.at[rc], add=True)
            return 0
        lax.fori_loop(0, NCHUNK, deg_chunk, 0)
        plsc.subcore_barrier()

        pltpu.sync_copy(acc_sh.at[pl.ds(s * RPT, RPT)], degb_v)

        def rsq(g, _):
            dloc_v[pl.ds(g * 16, 16)] = _rsqrt16(degb_v[pl.ds(g * 16, 16)])
            return 0
        lax.fori_loop(0, RPT // 16, rsq, 0)
        pltpu.sync_copy(
            dloc_v, dinv_hbm.at[pl.ds((c * 2 + h) * NPAD + s * RPT, RPT)])
        plsc.subcore_barrier()


def _sc_norms(rows):
    mesh = plsc.VectorSubcoreMesh(core_axis_name="c", subcore_axis_name="s",
                                  num_cores=NC, num_subcores=NS)
    return pl.kernel(
        _norm_body,
        out_type=jax.ShapeDtypeStruct((4 * NPAD,), jnp.float32),
        mesh=mesh,
        compiler_params=pltpu.CompilerParams(needs_layout_passes=False),
        scratch_types=[
            pltpu.VMEM_SHARED((NPAD,), jnp.float32),
            pltpu.VMEM((EPT,), jnp.int32),
            pltpu.VMEM((CH,), jnp.float32),
            pltpu.VMEM((RPT,), jnp.float32),
            pltpu.VMEM((RPT,), jnp.float32),
            pltpu.VMEM((RPT,), jnp.float32),
        ],
    )(rows)


SCH = 2000
NSCH = EPT // SCH
CPS = SCH // CH


def _agg_body(xs_hbm, rel_hbm, src_hbm, dst_hbm, et_hbm, agg_hbm,
              acc_sh, src_v, dst_v, et_v, xr0_v, rr0_v, xr1_v, rr1_v,
              ms0_v, ms1_v, sem0, sem1, ssem0, ssem1):
    c = lax.axis_index("c")
    s = lax.axis_index("s")
    zero16 = jnp.zeros((16,), jnp.float32)
    mhi = jnp.int32(-65536)
    xr = (xr0_v, xr1_v)
    rr = (rr0_v, rr1_v)
    ms = (ms0_v, ms1_v)
    sems = (sem0, sem1)
    ssems = (ssem0, ssem1)

    def issue(h, k, b):
        sc_i = src_v.at[pl.ds(k * CH, CH)]
        ec_i = et_v.at[pl.ds(k * CH, CH)]
        pltpu.async_copy(xs_hbm.at[c, h].at[sc_i], xr[b], sem=sems[b])
        pltpu.async_copy(rel_hbm.at[c].at[ec_i], rr[b], sem=sems[b])

    def wait(h, k, b):
        sc_i = src_v.at[pl.ds(k * CH, CH)]
        ec_i = et_v.at[pl.ds(k * CH, CH)]
        pltpu.make_async_copy(xs_hbm.at[c, h].at[sc_i], xr[b], sems[b]).wait()
        pltpu.make_async_copy(rel_hbm.at[c].at[ec_i], rr[b], sems[b]).wait()

    def compute_scatter(k, b):
        dc_i = dst_v.at[pl.ds(k * CH, CH)]

        @plsc.parallel_loop(0, CH, unroll=4)
        def edge(row):
            for g in range(4):
                sl = pl.ds(g * 16, 16)
                xw = xr[b][row, sl]
                rw = rr[b][row, sl]
                x_lo = lax.bitcast_convert_type(
                    lax.shift_left(xw, 16), jnp.float32)
                x_hi = lax.bitcast_convert_type(xw & mhi, jnp.float32)
                r_lo = lax.bitcast_convert_type(
                    lax.shift_left(rw, 16), jnp.float32)
                r_hi = lax.bitcast_convert_type(rw & mhi, jnp.float32)
                ms[b][row, pl.ds(g * 32, 16)] = x_lo * r_lo
                ms[b][row, pl.ds(g * 32 + 16, 16)] = x_hi * r_hi
        pltpu.async_copy(ms[b], acc_sh.at[dc_i], ssems[b], add=True)

    def wait_scat(k, b):
        dc_i = dst_v.at[pl.ds(k * CH, CH)]
        pltpu.make_async_copy(ms[b], acc_sh.at[dc_i], ssems[b]).wait()

    for h in range(2):
        def zinit(r, _):
            for j in range(8):
                ms0_v[r, pl.ds(j * 16, 16)] = zero16
            return 0
        lax.fori_loop(0, CH, zinit, 0)
        for t in range(8):
            pltpu.sync_copy(ms0_v, acc_sh.at[pl.ds(s * RPT + t * CH, CH)])
        plsc.subcore_barrier()

        off = (c * 2 + h) * EH + s * EPT

        def superchunk(u, _):
            soff = off + u * SCH
            pltpu.sync_copy(src_hbm.at[pl.ds(soff, SCH)], src_v)
            pltpu.sync_copy(dst_hbm.at[pl.ds(soff, SCH)], dst_v)
            pltpu.sync_copy(et_hbm.at[pl.ds(soff, SCH)], et_v)

            issue(h, 0, 0)

            def pair(k2, _):
                wait(h, 2 * k2, 0)
                issue(h, 2 * k2 + 1, 1)

                @pl.when(k2 > 0)
                def _():
                    wait_scat(2 * k2 - 2, 0)
                compute_scatter(2 * k2, 0)
                wait(h, 2 * k2 + 1, 1)
                issue(h, 2 * k2 + 2, 0)

                @pl.when(k2 > 0)
                def _():
                    wait_scat(2 * k2 - 1, 1)
                compute_scatter(2 * k2 + 1, 1)
                return 0
            lax.fori_loop(0, (CPS - 1) // 2, pair, 0)
            wait(h, CPS - 1, 0)
            wait_scat(CPS - 3, 0)
            compute_scatter(CPS - 1, 0)
            wait_scat(CPS - 2, 1)
            wait_scat(CPS - 1, 0)
            return 0
        lax.fori_loop(0, NSCH, superchunk, 0)
        plsc.subcore_barrier()

        pltpu.sync_copy(acc_sh.at[pl.ds(s * RPT, RPT)],
                        agg_hbm.at[c, h, pl.ds(s * RPT, RPT)])
        plsc.subcore_barrier()


def _sc_aggregate(xs, rel2, srcs, dsts, ets):
    mesh = plsc.VectorSubcoreMesh(core_axis_name="c", subcore_axis_name="s",
                                  num_cores=NC, num_subcores=NS)
    return pl.kernel(
        _agg_body,
        out_type=jax.ShapeDtypeStruct((2, 2, NPAD, D), jnp.float32),
        mesh=mesh,
        compiler_params=pltpu.CompilerParams(needs_layout_passes=False,
                                             use_tc_tiling_on_sc=False),
        scratch_types=[
            pltpu.VMEM_SHARED((NPAD, D), jnp.float32),
            pltpu.VMEM((SCH,), jnp.int32),
            pltpu.VMEM((SCH,), jnp.int32),
            pltpu.VMEM((SCH,), jnp.int32),
            pltpu.VMEM((CH, D // 2), jnp.int32),
            pltpu.VMEM((CH, D // 2), jnp.int32),
            pltpu.VMEM((CH, D // 2), jnp.int32),
            pltpu.VMEM((CH, D // 2), jnp.int32),
            pltpu.VMEM((CH, D), jnp.float32),
            pltpu.VMEM((CH, D), jnp.float32),
            pltpu.SemaphoreType.DMA,
            pltpu.SemaphoreType.DMA,
            pltpu.SemaphoreType.DMA,
            pltpu.SemaphoreType.DMA,
        ],
    )(xs, rel2, srcs, dsts, ets)


def _rel0_body(wt_ref, basis_ref, out_ref):
    for g in range(2):
        out_ref[g] = jnp.dot(wt_ref[g], basis_ref[g],
                             preferred_element_type=jnp.float32)


def _tc_rel0(wt2, basis2):
    return pl.pallas_call(
        _rel0_body,
        out_shape=jax.ShapeDtypeStruct((2, 32, D), jnp.float32),
    )(wt2, basis2)


def _relup_body(rel_ref, w_ref, out_ref):
    out_ref[0] = jnp.dot(rel_ref[0], w_ref[0],
                         preferred_element_type=jnp.float32)


def _tc_relup(rel_all2, w_rel2):
    return pl.pallas_call(
        _relup_body,
        grid=(2,),
        in_specs=[
            pl.BlockSpec((1, 33, D), lambda g: (g, 0, 0)),
            pl.BlockSpec((1, D, D), lambda g: (g, 0, 0)),
        ],
        out_specs=pl.BlockSpec((1, 33, D), lambda g: (g, 0, 0)),
        out_shape=jax.ShapeDtypeStruct((2, 33, D), jnp.float32),
    )(rel_all2, w_rel2)


def _make_pi():
    pi = []
    for g in range(4):
        pi += [g * 32 + 2 * l for l in range(16)]
        pi += [g * 32 + 2 * l + 1 for l in range(16)]
    return np.array(pi)


_PI = _make_pi()


def _pack_bf16(a):
    b = a.astype(jnp.bfloat16)
    return lax.bitcast_convert_type(
        b.reshape(*a.shape[:-1], a.shape[-1] // 2, 2), jnp.int32)


_RB = 2000


def _prescale_body(x_ref, di_ref, out_ref):
    x = x_ref[0]
    out_ref[0, 0] = x * di_ref[0, :, 0][:, None]
    out_ref[0, 1] = x * di_ref[0, :, 1][:, None]


def _tc_prescale(x2, dinv):
    return pl.pallas_call(
        _prescale_body,
        grid=(2, N // _RB),
        in_specs=[
            pl.BlockSpec((1, _RB, D), lambda g, r: (g, r, 0)),
            pl.BlockSpec((1, _RB, 2), lambda g, r: (g, r, 0)),
        ],
        out_specs=pl.BlockSpec((1, 2, _RB, D), lambda g, r: (g, 0, r, 0)),
        out_shape=jax.ShapeDtypeStruct((2, 2, N, D), jnp.float32),
    )(x2, dinv)


def _dense_body(ai_ref, ao_ref, x_ref, di_ref, rel_ref, wi_ref, wo_ref,
                wl_ref, out_ref):
    loop = rel_ref[0, 32, :][None, :]
    ai = ai_ref[0] * di_ref[0, :, 0][:, None]
    ao = ao_ref[0] * di_ref[0, :, 1][:, None]
    acc = jnp.dot(ai, wi_ref[0], preferred_element_type=jnp.float32)
    acc += jnp.dot(ao, wo_ref[0], preferred_element_type=jnp.float32)
    acc += jnp.dot(x_ref[0] * loop, wl_ref[0],
                   preferred_element_type=jnp.float32)
    out_ref[0] = jnp.tanh(acc * (1.0 / 3.0))


def _tc_dense(agg_in, agg_out, x2, dinv, rel_all2, wi2, wo2, wl2):
    row_spec = pl.BlockSpec((1, _RB, D), lambda g, r: (g, r, 0))
    w_spec = pl.BlockSpec((1, D, D), lambda g, r: (g, 0, 0))
    return pl.pallas_call(
        _dense_body,
        grid=(2, N // _RB),
        in_specs=[
            row_spec, row_spec, row_spec,
            pl.BlockSpec((1, _RB, 2), lambda g, r: (g, r, 0)),
            pl.BlockSpec((1, 33, D), lambda g, r: (g, 0, 0)),
            w_spec, w_spec, w_spec,
        ],
        out_specs=row_spec,
        out_shape=jax.ShapeDtypeStruct((2, N, D), jnp.float32),
    )(agg_in, agg_out, x2, dinv, rel_all2, wi2, wo2, wl2)


def kernel(params, user_edge_index, user_edge_type, item_edge_index,
           item_edge_type):
    p = params

    srcs = jnp.concatenate(
        [user_edge_index[0], item_edge_index[0]]).astype(jnp.int32)
    dsts = jnp.concatenate(
        [user_edge_index[1], item_edge_index[1]]).astype(jnp.int32)
    ets = jnp.concatenate(
        [user_edge_type, item_edge_type]).astype(jnp.int32)

    dinv = _sc_norms(srcs)
    dinv = jnp.transpose(
        dinv.reshape(2, 2, NPAD)[:, :, :N], (0, 2, 1))

    c1 = (p['u_conv1'], p['i_conv1'])
    c2 = (p['u_conv2'], p['i_conv2'])
    wt2 = jnp.stack([c1[0]['rel_wt'], c1[1]['rel_wt']])
    basis2 = jnp.stack([c1[0]['rel_basis'], c1[1]['rel_basis']])
    rel0 = _tc_rel0(wt2, basis2)
    loop1 = jnp.stack([c1[0]['loop_rel'], c1[1]['loop_rel']])
    rel_all1 = jnp.concatenate([rel0, loop1], axis=1)

    x0 = jnp.stack([p['emb_user'], p['emb_item']])

    def conv(x2, rel_all, cc):
        xs = _tc_prescale(x2, dinv)
        agg = _sc_aggregate(_pack_bf16(xs), _pack_bf16(rel_all),
                            srcs, dsts, ets)
        agg = agg[:, :, :N, :]
        wi2 = jnp.stack([cc[0]['w_in'], cc[1]['w_in']])[:, _PI, :]
        wo2 = jnp.stack([cc[0]['w_out'], cc[1]['w_out']])[:, _PI, :]
        wl2 = jnp.stack([cc[0]['w_loop'], cc[1]['w_loop']])
        return _tc_dense(agg[:, 0], agg[:, 1], x2, dinv, rel_all,
                         wi2, wo2, wl2)

    x1 = conv(x0, rel_all1, c1)

    wr2 = jnp.stack([c1[0]['w_rel'], c1[1]['w_rel']])
    relnew = _tc_relup(rel_all1, wr2)
    loop2 = jnp.stack([c2[0]['loop_rel'], c2[1]['loop_rel']])
    rel_all2 = jnp.concatenate([relnew[:, :32], loop2], axis=1)

    x2_out = conv(x1, rel_all2, c2)
    return (x2_out[0], x2_out[1])


# speedup vs baseline: 2.2909x; 2.2909x over previous
import functools

import jax
import jax.numpy as jnp
from jax import lax
from jax.experimental import pallas as pl
from jax.experimental.pallas import tpu as pltpu
from jax.experimental.pallas import tpu_sc as plsc

N = 10000
NPAD = 10240
D = 128
EH = 160000
NC = 2
NS = 16
EPT = EH // NS
CH = 80
NCHUNK = EPT // CH
RPT = NPAD // NS


def _bcast_lane(v16, lane):
    idx = jnp.full((16, 1), lane, jnp.int32)
    dnums = lax.GatherDimensionNumbers(
        offset_dims=(), collapsed_slice_dims=(0,), start_index_map=(0,))
    return lax.gather(v16, idx, dnums, (1,),
                      mode=lax.GatherScatterMode.PROMISE_IN_BOUNDS)


def _rsqrt16(x):
    i = lax.bitcast_convert_type(x, jnp.int32)
    i = jnp.int32(0x5F3759DF) - lax.shift_right_logical(i, 1)
    y = lax.bitcast_convert_type(i, jnp.float32)
    for _ in range(3):
        y = y * (1.5 - 0.5 * x * y * y)
    return jnp.where(x > 0.5, y, 0.0)


def _norm_body(rows_hbm, dinv_hbm,
               acc_sh, row_v, ones_v, zb_v, degb_v, dloc_v):
    c = lax.axis_index("c")
    s = lax.axis_index("s")
    zero16 = jnp.zeros((16,), jnp.float32)
    one16 = jnp.ones((16,), jnp.float32)

    def init_row(r, _):
        zb_v[pl.ds(r * 16, 16)] = zero16
        return 0
    lax.fori_loop(0, RPT // 16, init_row, 0)

    def init_ones(r, _):
        ones_v[pl.ds(r * 16, 16)] = one16
        return 0
    lax.fori_loop(0, CH // 16, init_ones, 0)

    for h in range(2):
        pltpu.sync_copy(zb_v, acc_sh.at[pl.ds(s * RPT, RPT)])
        plsc.subcore_barrier()

        off = (c * 2 + h) * EH + s * EPT
        pltpu.sync_copy(rows_hbm.at[pl.ds(off, EPT)], row_v)

        def deg_chunk(k, _):
            rc = row_v.at[pl.ds(k * CH, CH)]
            pltpu.sync_copy(ones_v, acc_sh.at[rc], add=True)
            return 0
        lax.fori_loop(0, NCHUNK, deg_chunk, 0)
        plsc.subcore_barrier()

        pltpu.sync_copy(acc_sh.at[pl.ds(s * RPT, RPT)], degb_v)

        def rsq(g, _):
            dloc_v[pl.ds(g * 16, 16)] = _rsqrt16(degb_v[pl.ds(g * 16, 16)])
            return 0
        lax.fori_loop(0, RPT // 16, rsq, 0)
        pltpu.sync_copy(
            dloc_v, dinv_hbm.at[pl.ds((c * 2 + h) * NPAD + s * RPT, RPT)])
        plsc.subcore_barrier()


def _sc_norms(rows):
    mesh = plsc.VectorSubcoreMesh(core_axis_name="c", subcore_axis_name="s",
                                  num_cores=NC, num_subcores=NS)
    return pl.kernel(
        _norm_body,
        out_type=jax.ShapeDtypeStruct((4 * NPAD,), jnp.float32),
        mesh=mesh,
        compiler_params=pltpu.CompilerParams(needs_layout_passes=False),
        scratch_types=[
            pltpu.VMEM_SHARED((NPAD,), jnp.float32),
            pltpu.VMEM((EPT,), jnp.int32),
            pltpu.VMEM((CH,), jnp.float32),
            pltpu.VMEM((RPT,), jnp.float32),
            pltpu.VMEM((RPT,), jnp.float32),
            pltpu.VMEM((RPT,), jnp.float32),
        ],
    )(rows)


SCH = 2000
NSCH = EPT // SCH
CPS = SCH // CH


def _agg_body(xs_hbm, rel_hbm, src_hbm, dst_hbm, et_hbm, agg_hbm,
              acc_sh, src_v, dst_v, et_v, relt_v, xr0_v, xr1_v,
              sem0, sem1, ssem0, ssem1):
    c = lax.axis_index("c")
    s = lax.axis_index("s")
    zero16 = jnp.zeros((16,), jnp.float32)
    iota16 = lax.iota(jnp.int32, 16)
    xr = (xr0_v, xr1_v)
    sems = (sem0, sem1)
    ssems = (ssem0, ssem1)

    pltpu.sync_copy(rel_hbm.at[c], relt_v)

    def issue(h, soff, k, b):
        sc_i = src_v.at[pl.ds(k * CH, CH)]
        pltpu.async_copy(xs_hbm.at[c, h].at[sc_i], xr[b], sem=sems[b])

    def wait(h, soff, k, b):
        sc_i = src_v.at[pl.ds(k * CH, CH)]
        pltpu.make_async_copy(xs_hbm.at[c, h].at[sc_i], xr[b], sems[b]).wait()

    def compute_scatter(k, b):
        dc_i = dst_v.at[pl.ds(k * CH, CH)]

        @plsc.parallel_loop(0, CH, unroll=4)
        def edge(row):
            e16 = et_v[pl.ds(k * CH + (row // 16) * 16, 16)]
            base = _bcast_lane(e16, row % 16) * 128 + iota16
            for j in range(8):
                sl = pl.ds(j * 16, 16)
                rv = plsc.load_gather(relt_v, [base + j * 16])
                xr[b][row, sl] = xr[b][row, sl] * rv
        pltpu.async_copy(xr[b], acc_sh.at[dc_i], ssems[b], add=True)

    def wait_scat(k, b):
        dc_i = dst_v.at[pl.ds(k * CH, CH)]
        pltpu.make_async_copy(xr[b], acc_sh.at[dc_i], ssems[b]).wait()

    for h in range(2):
        def zinit(r, _):
            for j in range(8):
                xr0_v[r, pl.ds(j * 16, 16)] = zero16
            return 0
        lax.fori_loop(0, CH, zinit, 0)
        for t in range(8):
            pltpu.sync_copy(xr0_v, acc_sh.at[pl.ds(s * RPT + t * CH, CH)])
        plsc.subcore_barrier()

        off = (c * 2 + h) * EH + s * EPT

        def superchunk(u, _):
            soff = off + u * SCH
            pltpu.sync_copy(src_hbm.at[pl.ds(soff, SCH)], src_v)
            pltpu.sync_copy(dst_hbm.at[pl.ds(soff, SCH)], dst_v)
            pltpu.sync_copy(et_hbm.at[pl.ds(soff, SCH)], et_v)

            issue(h, soff, 0, 0)

            def pair(k2, _):
                wait(h, soff, 2 * k2, 0)

                @pl.when(k2 > 0)
                def _():
                    wait_scat(2 * k2 - 1, 1)
                issue(h, soff, 2 * k2 + 1, 1)
                compute_scatter(2 * k2, 0)
                wait(h, soff, 2 * k2 + 1, 1)
                wait_scat(2 * k2, 0)
                issue(h, soff, 2 * k2 + 2, 0)
                compute_scatter(2 * k2 + 1, 1)
                return 0
            lax.fori_loop(0, (CPS - 1) // 2, pair, 0)
            wait(h, soff, CPS - 1, 0)
            wait_scat(CPS - 2, 1)
            compute_scatter(CPS - 1, 0)
            wait_scat(CPS - 1, 0)
            return 0
        lax.fori_loop(0, NSCH, superchunk, 0)
        plsc.subcore_barrier()

        pltpu.sync_copy(acc_sh.at[pl.ds(s * RPT, RPT)],
                        agg_hbm.at[c, h, pl.ds(s * RPT, RPT)])
        plsc.subcore_barrier()


def _sc_aggregate(xs, rel2, srcs, dsts, ets):
    mesh = plsc.VectorSubcoreMesh(core_axis_name="c", subcore_axis_name="s",
                                  num_cores=NC, num_subcores=NS)
    return pl.kernel(
        _agg_body,
        out_type=jax.ShapeDtypeStruct((2, 2, NPAD, D), jnp.float32),
        mesh=mesh,
        compiler_params=pltpu.CompilerParams(needs_layout_passes=False),
        scratch_types=[
            pltpu.VMEM_SHARED((NPAD, D), jnp.float32),
            pltpu.VMEM((SCH,), jnp.int32),
            pltpu.VMEM((SCH,), jnp.int32),
            pltpu.VMEM((SCH,), jnp.int32),
            pltpu.VMEM((33 * D,), jnp.float32),
            pltpu.VMEM((CH, D), jnp.float32),
            pltpu.VMEM((CH, D), jnp.float32),
            pltpu.SemaphoreType.DMA,
            pltpu.SemaphoreType.DMA,
            pltpu.SemaphoreType.DMA,
            pltpu.SemaphoreType.DMA,
        ],
    )(xs, rel2.reshape(2, 33 * D), srcs, dsts, ets)


def _rel0_body(wt_ref, basis_ref, out_ref):
    for g in range(2):
        out_ref[g] = jnp.dot(wt_ref[g], basis_ref[g],
                             preferred_element_type=jnp.float32)


def _tc_rel0(wt2, basis2):
    return pl.pallas_call(
        _rel0_body,
        out_shape=jax.ShapeDtypeStruct((2, 32, D), jnp.float32),
    )(wt2, basis2)


def _relup_body(rel_ref, w_ref, out_ref):
    out_ref[0] = jnp.dot(rel_ref[0], w_ref[0],
                         preferred_element_type=jnp.float32)


def _tc_relup(rel_all2, w_rel2):
    return pl.pallas_call(
        _relup_body,
        grid=(2,),
        in_specs=[
            pl.BlockSpec((1, 33, D), lambda g: (g, 0, 0)),
            pl.BlockSpec((1, D, D), lambda g: (g, 0, 0)),
        ],
        out_specs=pl.BlockSpec((1, 33, D), lambda g: (g, 0, 0)),
        out_shape=jax.ShapeDtypeStruct((2, 33, D), jnp.float32),
    )(rel_all2, w_rel2)


_RB = 2000


def _prescale_body(x_ref, di_ref, out_ref):
    x = x_ref[0]
    out_ref[0, 0] = x * di_ref[0, :, 0][:, None]
    out_ref[0, 1] = x * di_ref[0, :, 1][:, None]


def _tc_prescale(x2, dinv):
    return pl.pallas_call(
        _prescale_body,
        grid=(2, N // _RB),
        in_specs=[
            pl.BlockSpec((1, _RB, D), lambda g, r: (g, r, 0)),
            pl.BlockSpec((1, _RB, 2), lambda g, r: (g, r, 0)),
        ],
        out_specs=pl.BlockSpec((1, 2, _RB, D), lambda g, r: (g, 0, r, 0)),
        out_shape=jax.ShapeDtypeStruct((2, 2, N, D), jnp.float32),
    )(x2, dinv)


def _dense_body(ai_ref, ao_ref, x_ref, di_ref, rel_ref, wi_ref, wo_ref,
                wl_ref, out_ref):
    loop = rel_ref[0, 32, :][None, :]
    ai = ai_ref[0] * di_ref[0, :, 0][:, None]
    ao = ao_ref[0] * di_ref[0, :, 1][:, None]
    acc = jnp.dot(ai, wi_ref[0], preferred_element_type=jnp.float32)
    acc += jnp.dot(ao, wo_ref[0], preferred_element_type=jnp.float32)
    acc += jnp.dot(x_ref[0] * loop, wl_ref[0],
                   preferred_element_type=jnp.float32)
    out_ref[0] = jnp.tanh(acc * (1.0 / 3.0))


def _tc_dense(agg_in, agg_out, x2, dinv, rel_all2, wi2, wo2, wl2):
    row_spec = pl.BlockSpec((1, _RB, D), lambda g, r: (g, r, 0))
    w_spec = pl.BlockSpec((1, D, D), lambda g, r: (g, 0, 0))
    return pl.pallas_call(
        _dense_body,
        grid=(2, N // _RB),
        in_specs=[
            row_spec, row_spec, row_spec,
            pl.BlockSpec((1, _RB, 2), lambda g, r: (g, r, 0)),
            pl.BlockSpec((1, 33, D), lambda g, r: (g, 0, 0)),
            w_spec, w_spec, w_spec,
        ],
        out_specs=row_spec,
        out_shape=jax.ShapeDtypeStruct((2, N, D), jnp.float32),
    )(agg_in, agg_out, x2, dinv, rel_all2, wi2, wo2, wl2)


def kernel(params, user_edge_index, user_edge_type, item_edge_index,
           item_edge_type):
    p = params

    srcs = jnp.concatenate(
        [user_edge_index[0], item_edge_index[0]]).astype(jnp.int32)
    dsts = jnp.concatenate(
        [user_edge_index[1], item_edge_index[1]]).astype(jnp.int32)
    ets = jnp.concatenate(
        [user_edge_type, item_edge_type]).astype(jnp.int32)

    dinv = _sc_norms(srcs)
    dinv = jnp.transpose(
        dinv.reshape(2, 2, NPAD)[:, :, :N], (0, 2, 1))

    c1 = (p['u_conv1'], p['i_conv1'])
    c2 = (p['u_conv2'], p['i_conv2'])
    wt2 = jnp.stack([c1[0]['rel_wt'], c1[1]['rel_wt']])
    basis2 = jnp.stack([c1[0]['rel_basis'], c1[1]['rel_basis']])
    rel0 = _tc_rel0(wt2, basis2)
    loop1 = jnp.stack([c1[0]['loop_rel'], c1[1]['loop_rel']])
    rel_all1 = jnp.concatenate([rel0, loop1], axis=1)

    x0 = jnp.stack([p['emb_user'], p['emb_item']])

    def conv(x2, rel_all, cc):
        xs = _tc_prescale(x2, dinv)
        agg = _sc_aggregate(xs, rel_all, srcs, dsts, ets)
        agg = agg[:, :, :N, :]
        wi2 = jnp.stack([cc[0]['w_in'], cc[1]['w_in']])
        wo2 = jnp.stack([cc[0]['w_out'], cc[1]['w_out']])
        wl2 = jnp.stack([cc[0]['w_loop'], cc[1]['w_loop']])
        return _tc_dense(agg[:, 0], agg[:, 1], x2, dinv, rel_all,
                         wi2, wo2, wl2)

    x1 = conv(x0, rel_all1, c1)

    wr2 = jnp.stack([c1[0]['w_rel'], c1[1]['w_rel']])
    relnew = _tc_relup(rel_all1, wr2)
    loop2 = jnp.stack([c2[0]['loop_rel'], c2[1]['loop_rel']])
    rel_all2 = jnp.concatenate([relnew[:, :32], loop2], axis=1)

    x2_out = conv(x1, rel_all2, c2)
    return (x2_out[0], x2_out[1])
